# baseline (speedup 1.0000x reference)
import functools

import jax
import jax.numpy as jnp
from jax import lax
from jax.experimental import pallas as pl
from jax.experimental.pallas import tpu as pltpu
from jax.experimental.pallas import tpu_sc as plsc

N_NODES = 10000
N_EDGES = 320000
D = 128

NC = 2
NS = 16
NW = NC * NS

CHUNK = 128
EPW_CHUNKS = 80
EPW = EPW_CHUNKS * CHUNK
E_PAD = NW * EPW
AGG_ROWS = N_NODES
STRIPE = 624
LAST_STRIPE = AGG_ROWS - 15 * STRIPE


def _sc_segment_sum(x, src_w, dst_w, zeros):
  mesh = plsc.VectorSubcoreMesh(core_axis_name="c", subcore_axis_name="s")

  @functools.partial(
      pl.kernel,
      out_type=jax.ShapeDtypeStruct((NC, AGG_ROWS, D), jnp.float32),
      mesh=mesh,
      scratch_types=[
          pltpu.VMEM((EPW,), jnp.int32),
          pltpu.VMEM((EPW_CHUNKS, CHUNK), jnp.int32),
          pltpu.VMEM((CHUNK, D), jnp.float32),
          pltpu.VMEM_SHARED((AGG_ROWS, D), jnp.float32),
          pltpu.SemaphoreType.DMA,
          pltpu.SemaphoreType.DMA,
      ],
  )
  def k(x_hbm, src_hbm, dst_hbm, zeros_hbm, out_hbm,
        src_v, dst_v, rows_v, agg_sh, g0, g1):
    cid = lax.axis_index("c")
    sid = lax.axis_index("s")
    wid = sid * NC + cid

    @pl.when(sid < NS - 1)
    def _():
      pltpu.sync_copy(zeros_hbm.at[pl.ds(sid * STRIPE, STRIPE)],
                      agg_sh.at[pl.ds(sid * STRIPE, STRIPE)])

    @pl.when(sid == NS - 1)
    def _():
      pltpu.sync_copy(zeros_hbm.at[pl.ds(15 * STRIPE, LAST_STRIPE)],
                      agg_sh.at[pl.ds(15 * STRIPE, LAST_STRIPE)])

    pltpu.sync_copy(src_hbm.at[wid], src_v)
    pltpu.sync_copy(dst_hbm.at[wid], dst_v)
    plsc.subcore_barrier()

    def body(j, carry):
      pltpu.async_copy(x_hbm.at[src_v.at[pl.ds(j * CHUNK, CHUNK)]],
                       rows_v, g0).wait()
      pltpu.sync_copy(rows_v, agg_sh.at[dst_v.at[j]], add=True)
      return carry

    lax.fori_loop(0, EPW_CHUNKS, body, 0, unroll=False)

    plsc.subcore_barrier()

    @pl.when(sid < NS - 1)
    def _():
      pltpu.sync_copy(agg_sh.at[pl.ds(sid * STRIPE, STRIPE)],
                      out_hbm.at[cid, pl.ds(sid * STRIPE, STRIPE)])

    @pl.when(sid == NS - 1)
    def _():
      pltpu.sync_copy(agg_sh.at[pl.ds(15 * STRIPE, LAST_STRIPE)],
                      out_hbm.at[cid, pl.ds(15 * STRIPE, LAST_STRIPE)])

  return k(x, src_w, dst_w, zeros)


def _tc_linear(x, p, w1t, w2t, b2):
  blk = 1000

  def body(x_ref, p_ref, w1_ref, w2_ref, b_ref, o_ref):
    agg = p_ref[0] + p_ref[1]
    o_ref[...] = (
        jnp.dot(x_ref[...], w1_ref[...], preferred_element_type=jnp.float32)
        + jnp.dot(agg, w2_ref[...], preferred_element_type=jnp.float32)
        + b_ref[...]
    )

  return pl.pallas_call(
      body,
      grid=(N_NODES // blk,),
      in_specs=[
          pl.BlockSpec((blk, D), lambda i: (i, 0)),
          pl.BlockSpec((NC, blk, D), lambda i: (0, i, 0)),
          pl.BlockSpec((D, D), lambda i: (0, 0)),
          pl.BlockSpec((D, D), lambda i: (0, 0)),
          pl.BlockSpec((1, D), lambda i: (0, 0)),
      ],
      out_specs=pl.BlockSpec((blk, D), lambda i: (i, 0)),
      out_shape=jax.ShapeDtypeStruct((N_NODES, D), jnp.float32),
  )(x, p, w1t, w2t, b2)


@jax.jit
def kernel(x, edge_index, W, b):
  pad = E_PAD - N_EDGES
  x_pad = jnp.concatenate([x, jnp.zeros((8, D), jnp.float32)])
  src = jnp.concatenate([edge_index[0], jnp.full((pad,), N_NODES, jnp.int32)])
  dst = jnp.concatenate([edge_index[1], jnp.zeros((pad,), jnp.int32)])
  src_w = src.reshape(NW, EPW)
  dst_w = dst.reshape(NW, EPW_CHUNKS, CHUNK)
  zeros = jnp.zeros((AGG_ROWS, D), jnp.float32)
  p = _sc_segment_sum(x_pad, src_w, dst_w, zeros)
  w1t = W[:, :D].T
  w2t = W[:, D:].T
  return _tc_linear(x, p, w1t, w2t, b.reshape(1, D))


# speedup vs baseline: 1.6734x; 1.6734x over previous
import functools

import jax
import jax.numpy as jnp
from jax import lax
from jax.experimental import pallas as pl
from jax.experimental.pallas import tpu as pltpu
from jax.experimental.pallas import tpu_sc as plsc

N_NODES = 10000
N_EDGES = 320000
D = 128

NC = 2
NS = 16
NW = NC * NS

CHUNK = 128
C0 = 117
C1 = 41
E_PAD = NS * (C0 + C1) * CHUNK
AGG_ROWS = N_NODES
STRIPE = 624
LAST_STRIPE = AGG_ROWS - 15 * STRIPE


def _sc_segment_sum(x, src_w, dst_w, zeros):
  mesh = plsc.VectorSubcoreMesh(core_axis_name="c", subcore_axis_name="s")

  @functools.partial(
      pl.kernel,
      out_type=jax.ShapeDtypeStruct((NC, AGG_ROWS, D), jnp.float32),
      mesh=mesh,
      scratch_types=[
          pltpu.VMEM((C0, CHUNK), jnp.int32),
          pltpu.VMEM((C0, CHUNK), jnp.int32),
          pltpu.VMEM((CHUNK, D), jnp.float32),
          pltpu.VMEM_SHARED((AGG_ROWS, D), jnp.float32),
          pltpu.SemaphoreType.DMA,
      ],
  )
  def k(x_hbm, src_hbm, dst_hbm, zeros_hbm, out_hbm,
        src_v, dst_v, rows_v, agg_sh, g0):
    cid = lax.axis_index("c")
    sid = lax.axis_index("s")
    wid = cid * NS + sid
    nchunks = jnp.where(cid == 0, C0, C1)

    @pl.when(sid < NS - 1)
    def _():
      pltpu.sync_copy(zeros_hbm.at[pl.ds(sid * STRIPE, STRIPE)],
                      agg_sh.at[pl.ds(sid * STRIPE, STRIPE)])

    @pl.when(sid == NS - 1)
    def _():
      pltpu.sync_copy(zeros_hbm.at[pl.ds(15 * STRIPE, LAST_STRIPE)],
                      agg_sh.at[pl.ds(15 * STRIPE, LAST_STRIPE)])

    pltpu.sync_copy(src_hbm.at[wid], src_v)
    pltpu.sync_copy(dst_hbm.at[wid], dst_v)
    plsc.subcore_barrier()

    def body(j, carry):
      pltpu.async_copy(x_hbm.at[src_v.at[j]], rows_v, g0).wait()
      pltpu.sync_copy(rows_v, agg_sh.at[dst_v.at[j]], add=True)
      return carry

    lax.fori_loop(0, nchunks, body, 0, unroll=False)

    plsc.subcore_barrier()

    @pl.when(sid < NS - 1)
    def _():
      pltpu.sync_copy(agg_sh.at[pl.ds(sid * STRIPE, STRIPE)],
                      out_hbm.at[cid, pl.ds(sid * STRIPE, STRIPE)])

    @pl.when(sid == NS - 1)
    def _():
      pltpu.sync_copy(agg_sh.at[pl.ds(15 * STRIPE, LAST_STRIPE)],
                      out_hbm.at[cid, pl.ds(15 * STRIPE, LAST_STRIPE)])

  return k(x, src_w, dst_w, zeros)


def _tc_linear(x, p, w1t, w2t, b2):
  blk = 1000

  def body(x_ref, p_ref, w1_ref, w2_ref, b_ref, o_ref):
    agg = p_ref[0] + p_ref[1]
    o_ref[...] = (
        jnp.dot(x_ref[...], w1_ref[...], preferred_element_type=jnp.float32)
        + jnp.dot(agg, w2_ref[...], preferred_element_type=jnp.float32)
        + b_ref[...]
    )

  return pl.pallas_call(
      body,
      grid=(N_NODES // blk,),
      in_specs=[
          pl.BlockSpec((blk, D), lambda i: (i, 0)),
          pl.BlockSpec((NC, blk, D), lambda i: (0, i, 0)),
          pl.BlockSpec((D, D), lambda i: (0, 0)),
          pl.BlockSpec((D, D), lambda i: (0, 0)),
          pl.BlockSpec((1, D), lambda i: (0, 0)),
      ],
      out_specs=pl.BlockSpec((blk, D), lambda i: (i, 0)),
      out_shape=jax.ShapeDtypeStruct((N_NODES, D), jnp.float32),
  )(x, p, w1t, w2t, b2)


def _split_chunks(a, fill):
  e0 = NS * C0 * CHUNK
  r0 = a[:e0].reshape(NS, C0, CHUNK)
  r1 = a[e0:].reshape(NS, C1, CHUNK)
  r1 = jnp.concatenate(
      [r1, jnp.full((NS, C0 - C1, CHUNK), fill, jnp.int32)], axis=1)
  return jnp.concatenate([r0, r1])


@jax.jit
def kernel(x, edge_index, W, b):
  pad = E_PAD - N_EDGES
  x_pad = jnp.concatenate([x, jnp.zeros((8, D), jnp.float32)])
  src = jnp.concatenate([edge_index[0], jnp.full((pad,), N_NODES, jnp.int32)])
  dst = jnp.concatenate([edge_index[1], jnp.zeros((pad,), jnp.int32)])
  src_w = _split_chunks(src, N_NODES)
  dst_w = _split_chunks(dst, 0)
  zeros = jnp.zeros((AGG_ROWS, D), jnp.float32)
  p = _sc_segment_sum(x_pad, src_w, dst_w, zeros)
  w1t = W[:, :D].T
  w2t = W[:, D:].T
  return _tc_linear(x, p, w1t, w2t, b.reshape(1, D))
